# Initial kernel scaffold; baseline (speedup 1.0000x reference)
import jax
import jax.numpy as jnp
from jax.experimental import pallas as pl


def kernel(x, edge_index, edge_type, params):
    raise NotImplementedError("write your pallas kernel here")


# speedup vs baseline: 3.9623x; 3.9623x over previous
import functools

import jax
import jax.numpy as jnp
from jax import lax
from jax.experimental import pallas as pl
from jax.experimental.pallas import tpu as pltpu
from jax.experimental.pallas import tpu_sc as plsc

N = 10000
E = 160000
D = 256
R = 8

NC = 2
NS = 16
L = 16

NB = N * R
PB = 81920
PBROWS = PB // L
EP = 160256
EPT = EP // (NC * NS)
BLK = 128
G = 2560
NG = 4
EC4 = 160000
DUMMY_BIN = NB

_mesh = plsc.VectorSubcoreMesh(core_axis_name="c", subcore_axis_name="s")


def _iota16():
    return lax.iota(jnp.int32, 16)


_GDN = lax.GatherDimensionNumbers(
    offset_dims=(), collapsed_slice_dims=(0,), start_index_map=(0,))


def _take16(vec, idx_vec):
    return lax.gather(vec, idx_vec[:, None], _GDN, (1,),
                      mode=lax.GatherScatterMode.PROMISE_IN_BOUNDS)


NT = NC * NS


@functools.partial(
    pl.kernel,
    mesh=_mesh,
    out_type=[jax.ShapeDtypeStruct((NT, PB), jnp.float32)],
    scratch_types=[
        pltpu.VMEM((EPT,), jnp.int32),
        pltpu.VMEM((PB,), jnp.float32),
    ],
    compiler_params=pltpu.CompilerParams(needs_layout_passes=False),
)
def _k_hist(q_hbm, zeros_hbm, hist_hbm, qbuf, histv):
    c = lax.axis_index("c")
    s = lax.axis_index("s")
    w = c * NS + s
    pltpu.sync_copy(zeros_hbm, histv)
    pltpu.sync_copy(q_hbm.at[pl.ds(w * EPT, EPT)], qbuf)
    onev = jnp.full((L,), 1.0, jnp.float32)
    mask0 = _iota16() == jnp.full((L,), 0, jnp.int32)

    def body(j, carry):
        jv = jnp.full((L,), j, jnp.int32)
        qj = plsc.load_gather(qbuf, [jv])
        plsc.addupdate_scatter(histv, [qj], onev, mask=mask0)
        return carry

    lax.fori_loop(0, EPT, body, 0)
    pltpu.sync_copy(histv, hist_hbm.at[w])


_K2C = PB // NT


@functools.partial(
    pl.kernel,
    mesh=_mesh,
    out_type=[jax.ShapeDtypeStruct((PB,), jnp.float32)],
    scratch_types=[
        pltpu.VMEM((_K2C,), jnp.float32),
        pltpu.VMEM((_K2C,), jnp.float32),
        pltpu.VMEM((_K2C,), jnp.float32),
    ],
)
def _k_invcnt(hist_hbm, inv_hbm, abuf, bbuf, obuf):
    c = lax.axis_index("c")
    s = lax.axis_index("s")
    off = (c * NS + s) * _K2C
    pltpu.sync_copy(hist_hbm.at[0, pl.ds(off, _K2C)], abuf)
    for t in range(1, NT):
        pltpu.sync_copy(hist_hbm.at[t, pl.ds(off, _K2C)], bbuf)

        def addbody(i, carry):
            abuf[pl.ds(i * L, L)] = abuf[pl.ds(i * L, L)] + bbuf[pl.ds(i * L, L)]
            return carry

        lax.fori_loop(0, _K2C // L, addbody, 0)

    onev = jnp.full((L,), 1.0, jnp.float32)

    def body(i, carry):
        va = abuf[pl.ds(i * L, L)]
        obuf[pl.ds(i * L, L)] = onev / jnp.maximum(va, onev)
        return carry

    lax.fori_loop(0, _K2C // L, body, 0)
    pltpu.sync_copy(obuf, inv_hbm.at[pl.ds(off, _K2C)])


@functools.partial(
    pl.kernel,
    mesh=_mesh,
    out_type=[jax.ShapeDtypeStruct((EP,), jnp.float32)],
    scratch_types=[
        pltpu.VMEM((PB,), jnp.float32),
        pltpu.VMEM((EPT,), jnp.int32),
        pltpu.VMEM((EPT,), jnp.float32),
    ],
    compiler_params=pltpu.CompilerParams(needs_layout_passes=False),
)
def _k_scale(inv_hbm, q_hbm, scale_hbm, invbuf, qbuf, obuf):
    c = lax.axis_index("c")
    s = lax.axis_index("s")
    base = (c * NS + s) * EPT
    pltpu.sync_copy(inv_hbm, invbuf)
    pltpu.sync_copy(q_hbm.at[pl.ds(base, EPT)], qbuf)

    def body(i, carry):
        qv = qbuf[pl.ds(i * L, L)]
        obuf[pl.ds(i * L, L)] = plsc.load_gather(invbuf, [qv])
        return carry

    lax.fori_loop(0, EPT // L, body, 0)
    pltpu.sync_copy(obuf, scale_hbm.at[pl.ds(base, EPT)])


GT = 320
BLK = 64
EC32 = 160000


@functools.partial(
    pl.kernel,
    mesh=_mesh,
    out_type=[jax.ShapeDtypeStruct((NT * GT, D), jnp.float32)],
    scratch_types=[
        pltpu.VMEM((BLK,), jnp.int32),
        pltpu.VMEM((BLK,), jnp.int32),
        pltpu.VMEM((BLK,), jnp.float32),
        pltpu.VMEM((BLK, D), jnp.float32),
        pltpu.VMEM((L,), jnp.int32),
        pltpu.VMEM((GT, D), jnp.float32),
        pltpu.SemaphoreType.DMA,
    ],
    compiler_params=pltpu.CompilerParams(needs_layout_passes=False),
)
def _k_edge(ytab_hbm, gidx_hbm, loc_hbm, scale_hbm, nblk_hbm, zeros_hbm,
            agg_hbm, gbuf, lbuf, sbuf, rowbuf, nbv, acc, sem):
    c = lax.axis_index("c")
    s = lax.axis_index("s")
    w = c * NS + s

    pltpu.sync_copy(zeros_hbm, acc)
    pltpu.sync_copy(nblk_hbm.at[w], nbv)
    trips = nbv[...][0]

    colvs = [jnp.arange(k * L, k * L + L, dtype=jnp.int32)
             for k in range(D // L)]

    def body(i, carry):
        off = i * BLK
        pltpu.sync_copy(gidx_hbm.at[w, pl.ds(off, BLK)], gbuf)
        pltpu.sync_copy(loc_hbm.at[w, pl.ds(off, BLK)], lbuf)
        pltpu.sync_copy(scale_hbm.at[w, pl.ds(off, BLK)], sbuf)
        pltpu.async_copy(ytab_hbm.at[gbuf], rowbuf, sem).wait()

        def rbody(j, rc):
            jv = jnp.full((L,), j, jnp.int32)
            bj = plsc.load_gather(sbuf, [jv])
            locv = plsc.load_gather(lbuf, [jv])
            for k in range(D // L):
                val = rowbuf[j, pl.ds(k * L, L)] * bj
                plsc.addupdate_scatter(acc, [locv, colvs[k]], val)
            return rc

        lax.fori_loop(0, BLK, rbody, 0)
        return carry

    lax.fori_loop(0, trips, body, 0)
    pltpu.sync_copy(acc, agg_hbm.at[pl.ds(w * GT, GT)])


RB = 200
NBLOCKS = N // RB


def _dense_math(x, wall_ref, root_ref, bias_ref, w1_ref, b1_ref, g_ref,
                be_ref, w2_ref, b2_ref, y_ref, base_ref):
    y_ref[...] = jnp.dot(x, wall_ref[...], preferred_element_type=jnp.float32)
    base = jnp.dot(x, root_ref[...], preferred_element_type=jnp.float32)
    base = base + bias_ref[...]
    h1 = jnp.dot(x, w1_ref[...], preferred_element_type=jnp.float32) + b1_ref[...]
    mu = jnp.mean(h1, axis=-1, keepdims=True)
    var = jnp.mean((h1 - mu) ** 2, axis=-1, keepdims=True)
    h1 = (h1 - mu) * lax.rsqrt(var + 1e-5) * g_ref[...] + be_ref[...]
    h1 = jnp.maximum(h1, 0.0)
    base = base + jnp.dot(h1, w2_ref[...], preferred_element_type=jnp.float32)
    base_ref[...] = base + b2_ref[...]


def _tc_first_body(x_ref, *rest):
    _dense_math(x_ref[...], *rest)


def _tc_fused_body(agg_ref, prev_ref, *rest):
    x = jnp.maximum(agg_ref[...] + prev_ref[...], 0.0)
    _dense_math(x, *rest)


_w_specs = [
    pl.BlockSpec((D, R * D), lambda i: (0, 0)),
    pl.BlockSpec((D, D), lambda i: (0, 0)),
    pl.BlockSpec((D,), lambda i: (0,)),
    pl.BlockSpec((D, D), lambda i: (0, 0)),
    pl.BlockSpec((D,), lambda i: (0,)),
    pl.BlockSpec((D,), lambda i: (0,)),
    pl.BlockSpec((D,), lambda i: (0,)),
    pl.BlockSpec((D, D), lambda i: (0, 0)),
    pl.BlockSpec((D,), lambda i: (0,)),
]
_row_spec = pl.BlockSpec((RB, D), lambda i: (i, 0))
_agg_spec = pl.BlockSpec((RB, D), lambda i: (i, 0))
_dense_out = [
    jax.ShapeDtypeStruct((N, R * D), jnp.float32),
    jax.ShapeDtypeStruct((N, D), jnp.float32),
]
_dense_out_specs = [pl.BlockSpec((RB, R * D), lambda i: (i, 0)), _row_spec]

_tc_first = pl.pallas_call(
    _tc_first_body,
    grid=(NBLOCKS,),
    in_specs=[_row_spec] + _w_specs,
    out_specs=_dense_out_specs,
    out_shape=_dense_out,
)

_tc_fused = pl.pallas_call(
    _tc_fused_body,
    grid=(NBLOCKS,),
    in_specs=[_agg_spec, _row_spec] + _w_specs,
    out_specs=_dense_out_specs,
    out_shape=_dense_out,
)


def _tc_final_body(agg_ref, prev_ref, o_ref):
    o_ref[...] = agg_ref[...] + prev_ref[...]

_tc_final = pl.pallas_call(
    _tc_final_body,
    grid=(NBLOCKS,),
    in_specs=[_agg_spec, _row_spec],
    out_specs=_row_spec,
    out_shape=jax.ShapeDtypeStruct((N, D), jnp.float32),
)


def _layer_weights(p):
    w = (p['comp'] @ p['weight'].reshape(p['weight'].shape[0], -1))
    w = w.reshape(R, D, D)
    wall = w.transpose(1, 0, 2).reshape(D, R * D)
    e = 1.0 + p['eps'][0]
    return (wall, p['root'], p['bias'], e * p['mlp_w1'], p['mlp_b1'],
            p['ln_g'], p['ln_b'], p['mlp_w2'], p['mlp_b2'])


def kernel(x, edge_index, edge_type, params):
    src = edge_index[0].astype(jnp.int32)
    dst = edge_index[1].astype(jnp.int32)
    et = edge_type.astype(jnp.int32)

    q = dst * R + et
    gidx = src * R + et
    q_pad = jnp.concatenate([q, jnp.full((EP - E,), DUMMY_BIN, jnp.int32)])

    zeros_hist = jnp.zeros((PB,), jnp.float32)
    zeros_out = jnp.zeros((GT, D), jnp.float32)

    (hist,) = _k_hist(q_pad, zeros_hist)
    (inv,) = _k_invcnt(hist)
    (scale_all,) = _k_scale(inv, q_pad)
    scale_e = scale_all[:E]

    grp = dst // GT
    pos = jnp.zeros((E,), jnp.int32)
    cnts = []
    for g in range(NT):
        mg = (grp == g)
        cg = jnp.cumsum(mg.astype(jnp.int32))
        pos = jnp.where(mg, cg - 1, pos)
        cnts.append(cg[-1])
    gidx_p = jnp.zeros((NT, EC32), jnp.int32).at[grp, pos].set(gidx)
    loc_p = jnp.zeros((NT, EC32), jnp.int32).at[grp, pos].set(dst - grp * GT)
    scale_p = jnp.zeros((NT, EC32), jnp.float32).at[grp, pos].set(scale_e)
    nblk = (jnp.stack(cnts) + BLK - 1) // BLK
    nblk_b = jnp.broadcast_to(nblk.astype(jnp.int32)[:, None], (NT, L))

    h = x
    base = None
    agg = None
    for li, lname in enumerate(('l1', 'l2', 'l3')):
        wts = _layer_weights(params[lname])
        if li == 0:
            y, base = _tc_first(h, *wts)
        else:
            y, base = _tc_fused(agg, base, *wts)
        ytab = y.reshape(N * R, D)
        (agg,) = _k_edge(ytab, gidx_p, loc_p, scale_p, nblk_b, zeros_out)

    return _tc_final(agg, base)
